# Initial kernel scaffold; baseline (speedup 1.0000x reference)
import jax
import jax.numpy as jnp
from jax.experimental import pallas as pl


def kernel(h_s, A_vs, v_embed, W_sv, b_sv, W_vs, b_vs):
    raise NotImplementedError("write your pallas kernel here")


# speedup vs baseline: 2.8514x; 2.8514x over previous
import functools

import jax
import jax.numpy as jnp
from jax import lax
from jax.experimental import pallas as pl
from jax.experimental.pallas import tpu as pltpu
from jax.experimental.pallas import tpu_sc as plsc

NV = 10000
NS = 10000
D = 128
NL = 3
E = 320000

NCORE = 2
NTILE = 16
CHUNK = 128
NCHUNK = 157
EPT = NCHUNK * CHUNK
EPAD = NTILE * EPT
NROW = NV + 112
RPT = NROW // NTILE

_mesh = plsc.VectorSubcoreMesh(core_axis_name="c", subcore_axis_name="s")


HALF = D // 2

@functools.partial(
    pl.kernel,
    out_type=jax.ShapeDtypeStruct((2 * NCORE * NROW, HALF), jnp.float32),
    mesh=_mesh,
    scratch_types=[
        pltpu.VMEM((NCHUNK, CHUNK), jnp.int32),
        pltpu.VMEM((NCHUNK, CHUNK), jnp.int32),
        pltpu.VMEM((NCHUNK, CHUNK), jnp.int32),
        pltpu.VMEM((CHUNK, HALF), jnp.float32),
        pltpu.VMEM_SHARED((NROW, HALF), jnp.float32),
        pltpu.SemaphoreType.DMA,
    ],
    compiler_params=pltpu.CompilerParams(use_tc_tiling_on_sc=False),
)
def _sc_aggregate(table_hbm, src_hbm, dst_hbm, zeros_hbm, out_hbm,
                  srcA_v, srcB_v, dst_v, rows_v, accum, sem):
    c = lax.axis_index("c")
    s = lax.axis_index("s")
    wid = c * NTILE + s

    pltpu.sync_copy(src_hbm.at[wid], srcA_v)
    pltpu.sync_copy(src_hbm.at[NCORE * NTILE + wid], srcB_v)
    pltpu.sync_copy(dst_hbm.at[wid], dst_v)

    for h, src_v in ((0, srcA_v), (1, srcB_v)):
        pltpu.sync_copy(zeros_hbm.at[pl.ds(s * RPT, RPT)],
                        accum.at[pl.ds(s * RPT, RPT)])
        plsc.subcore_barrier()

        def body(i, carry):
            pltpu.async_copy(table_hbm.at[src_v.at[i]], rows_v, sem).wait()
            pltpu.sync_copy(rows_v, accum.at[dst_v.at[i]], add=True)
            return carry

        lax.fori_loop(0, NCHUNK, body, 0, unroll=False)
        plsc.subcore_barrier()

        pltpu.sync_copy(
            accum.at[pl.ds(s * RPT, RPT)],
            out_hbm.at[pl.ds((h * NCORE + c) * NROW + s * RPT, RPT)])


DEGW = 64

@functools.partial(
    pl.kernel,
    out_type=jax.ShapeDtypeStruct((NCORE * NROW, DEGW), jnp.float32),
    mesh=_mesh,
    scratch_types=[
        pltpu.VMEM((NCHUNK, CHUNK), jnp.int32),
        pltpu.VMEM((CHUNK, DEGW), jnp.float32),
        pltpu.VMEM_SHARED((NROW, DEGW), jnp.float32),
    ],
    compiler_params=pltpu.CompilerParams(use_tc_tiling_on_sc=False),
)
def _sc_degrees(dst_hbm, ones_hbm, zeros_hbm, out_hbm, dst_v, ones_v, accum):
    c = lax.axis_index("c")
    s = lax.axis_index("s")
    wid = c * NTILE + s

    pltpu.sync_copy(dst_hbm.at[wid], dst_v)
    pltpu.sync_copy(ones_hbm, ones_v)
    pltpu.sync_copy(zeros_hbm.at[pl.ds(s * RPT, RPT)],
                    accum.at[pl.ds(s * RPT, RPT)])
    plsc.subcore_barrier()

    def body(i, carry):
        pltpu.sync_copy(ones_v, accum.at[dst_v.at[i]], add=True)
        return carry

    lax.fori_loop(0, NCHUNK, body, 0, unroll=False)
    plsc.subcore_barrier()

    pltpu.sync_copy(accum.at[pl.ds(s * RPT, RPT)],
                    out_hbm.at[pl.ds(c * NROW + s * RPT, RPT)])


BLK = 400
NBLK = NV // BLK


def _tc_body(relu, hv_ref, av_ref, dv_ref, hs_ref, as_ref, ds_ref,
             wv_ref, bv_ref, ws_ref, bs_ref, ov_ref, os_ref):
    dv = jnp.maximum(dv_ref[:, 0:1], 1.0)
    xs = hv_ref[...] + av_ref[...] / dv
    ov = jnp.dot(xs, wv_ref[...], preferred_element_type=jnp.float32) + bv_ref[...]
    ds_ = jnp.maximum(ds_ref[:, 0:1], 1.0)
    ys = hs_ref[...] + as_ref[...] / ds_
    os_ = jnp.dot(ys, ws_ref[...], preferred_element_type=jnp.float32) + bs_ref[...]
    if relu:
        ov = jnp.maximum(ov, 0.0)
        os_ = jnp.maximum(os_, 0.0)
    ov_ref[...] = ov
    os_ref[...] = os_


def _tc_layer(relu, hv, aggv, degv, hs, aggs, degs, wv, bv, ws, bs):
    row = pl.BlockSpec((BLK, D), lambda i: (i, 0))
    deg = pl.BlockSpec((BLK, DEGW), lambda i: (i, 0))
    full = pl.BlockSpec((D, D), lambda i: (0, 0))
    bias = pl.BlockSpec((1, D), lambda i: (0, 0))
    return pl.pallas_call(
        functools.partial(_tc_body, relu),
        grid=(NBLK,),
        in_specs=[row, row, deg, row, row, deg, full, bias, full, bias],
        out_specs=[row, row],
        out_shape=[jax.ShapeDtypeStruct((NV, D), jnp.float32),
                   jax.ShapeDtypeStruct((NS, D), jnp.float32)],
    )(hv, aggv, degv, hs, aggs, degs, wv, bv, ws, bs)


def kernel(h_s, A_vs, v_embed, W_sv, b_sv, W_vs, b_vs):
    v_idx = A_vs[0].astype(jnp.int32)
    s_idx = A_vs[1].astype(jnp.int32)

    npad = EPAD - E
    pad0 = jnp.zeros((npad,), jnp.int32)
    padd = jnp.full((npad,), NV, jnp.int32)
    base = jnp.stack([
        jnp.concatenate([s_idx, pad0]),
        jnp.concatenate([v_idx + NV, pad0]),
    ])
    src = jnp.concatenate([2 * base, 2 * base + 1]).reshape(
        2 * NCORE * NTILE, NCHUNK, CHUNK)
    dst = jnp.stack([
        jnp.concatenate([v_idx, padd]),
        jnp.concatenate([s_idx, padd]),
    ]).reshape(NCORE * NTILE, NCHUNK, CHUNK)

    zeros = jnp.zeros((NROW, HALF), jnp.float32)
    zeros8 = jnp.zeros((NROW, DEGW), jnp.float32)
    ones8 = jnp.ones((CHUNK, DEGW), jnp.float32)

    deg = _sc_degrees(dst, ones8, zeros8)
    degv = deg[:NV]
    degs = deg[NROW:NROW + NS]

    hv, hs = v_embed, h_s
    for l in range(NL):
        table = jnp.concatenate([hs, hv], axis=0).reshape(-1, HALF)
        out = _sc_aggregate(table, src, dst, zeros)
        agg = out.reshape(2, NCORE, NROW, HALF).transpose(1, 2, 0, 3)
        agg = agg.reshape(NCORE, NROW, D)
        aggv = agg[0, :NV]
        aggs = agg[1, :NS]
        hv, hs = _tc_layer(l < NL - 1, hv, aggv, degv, hs, aggs, degs,
                           W_sv[l], b_sv[l].reshape(1, D),
                           W_vs[l], b_vs[l].reshape(1, D))
    return hv
